# Initial kernel scaffold; baseline (speedup 1.0000x reference)
import functools

import jax
import jax.numpy as jnp
from jax import lax
from jax.experimental import pallas as pl
from jax.experimental.pallas import tpu as pltpu
from jax.experimental.pallas import tpu_sc as plsc

VOCAB = 1000
VPAD = 1024
BATCH = 16384
SEQ = 200
DIM = 64

NC = 2
NS = 16
NW = NC * NS
LANES = 16
ROWS_PER_TILE = BATCH // NW
GROUPS_PER_TILE = ROWS_PER_TILE // LANES


def _proj_kernel(t_ref, wt_ref, b_ref, o_ref):
    o_ref[...] = (
        jnp.dot(t_ref[...], wt_ref[...], preferred_element_type=jnp.float32)
        + b_ref[...]
    )


def _sc_body(v_hbm, x_hbm, out_hbm, v_v, x_v, out_v):
    wid = lax.axis_index("s") * NC + lax.axis_index("c")
    row0 = wid * ROWS_PER_TILE
    base = row0 * SEQ

    pltpu.sync_copy(v_hbm, v_v)
    pltpu.sync_copy(x_hbm.at[pl.ds(base, ROWS_PER_TILE * SEQ)], x_v)

    lane_off = lax.iota(jnp.int32, (16,)) * SEQ

    def group_body(g, _):
        gbase = g * (LANES * SEQ)

        def tok_body(t, acc):
            offs = lane_off + (gbase + t)
            idxs = plsc.load_gather(x_v, [offs])
            vals = plsc.load_gather(v_v, [idxs])
            return acc + vals

        acc = lax.fori_loop(0, SEQ, tok_body, jnp.zeros((16,), jnp.float32))
        z = acc * (1.0 / SEQ)
        y = 1.0 / (1.0 + jnp.exp(-z))
        out_v[pl.ds(g * LANES, LANES)] = y
        return 0

    lax.fori_loop(0, GROUPS_PER_TILE, group_body, 0)

    pltpu.sync_copy(out_v, out_hbm.at[pl.ds(row0, ROWS_PER_TILE)])


_sc_call = functools.partial(
    pl.kernel,
    out_type=jax.ShapeDtypeStruct((BATCH,), jnp.float32),
    mesh=plsc.VectorSubcoreMesh(core_axis_name="c", subcore_axis_name="s"),
    scratch_types=[
        pltpu.VMEM((VPAD,), jnp.float32),
        pltpu.VMEM((ROWS_PER_TILE * SEQ,), jnp.int32),
        pltpu.VMEM((ROWS_PER_TILE,), jnp.float32),
    ],
)(_sc_body)


def kernel(x, emb_table, W, b):
    table_pad = jnp.pad(emb_table, ((0, VPAD - VOCAB), (0, 0)))
    wt = W.reshape(DIM, 1)
    b2 = b.reshape(1, 1)
    v = pl.pallas_call(
        _proj_kernel,
        out_shape=jax.ShapeDtypeStruct((VPAD, 1), jnp.float32),
    )(table_pad, wt, b2)
    out = _sc_call(v.reshape(VPAD), x.reshape(BATCH * SEQ))
    return out.reshape(BATCH, 1)


# speedup vs baseline: 126.9805x; 126.9805x over previous
import functools

import jax
import jax.numpy as jnp
from jax import lax
from jax.experimental import pallas as pl
from jax.experimental.pallas import tpu as pltpu
from jax.experimental.pallas import tpu_sc as plsc

VOCAB = 1000
VPAD = 1024
BATCH = 16384
SEQ = 200
DIM = 64

NC = 2
NS = 16
NW = NC * NS
LANES = 16
ROWS_PER_TILE = BATCH // NW
GROUPS_PER_TILE = ROWS_PER_TILE // LANES


def _proj_kernel(t_ref, wt_ref, b_ref, o_ref):
    o_ref[...] = (
        jnp.dot(t_ref[...], wt_ref[...], preferred_element_type=jnp.float32)
        + b_ref[...]
    )


def _sc_body(v_hbm, x_hbm, out_hbm, v_v, x_v, out_v):
    wid = lax.axis_index("s") * NC + lax.axis_index("c")
    row0 = wid * ROWS_PER_TILE
    base = row0 * SEQ

    pltpu.sync_copy(v_hbm, v_v)
    pltpu.sync_copy(x_hbm.at[pl.ds(base, ROWS_PER_TILE * SEQ)], x_v)

    lane_off = lax.iota(jnp.int32, 16) * SEQ

    def group_body(g, _):
        gbase = g * (LANES * SEQ)

        def tok_body(t, acc):
            offs = lane_off + (gbase + t)
            idxs = plsc.load_gather(x_v, [offs])
            vals = plsc.load_gather(v_v, [idxs])
            return acc + vals

        acc = lax.fori_loop(0, SEQ, tok_body, jnp.zeros((16,), jnp.float32))
        z = acc * (1.0 / SEQ)
        y = 1.0 / (1.0 + jnp.exp(-z))
        out_v[pl.ds(g * LANES, LANES)] = y
        return 0

    lax.fori_loop(0, GROUPS_PER_TILE, group_body, 0)

    pltpu.sync_copy(out_v, out_hbm.at[pl.ds(row0, ROWS_PER_TILE)])


_sc_call = functools.partial(
    pl.kernel,
    out_type=jax.ShapeDtypeStruct((BATCH,), jnp.float32),
    mesh=plsc.VectorSubcoreMesh(core_axis_name="c", subcore_axis_name="s"),
    scratch_types=[
        pltpu.VMEM((VPAD,), jnp.float32),
        pltpu.VMEM((ROWS_PER_TILE * SEQ,), jnp.int32),
        pltpu.VMEM((ROWS_PER_TILE,), jnp.float32),
    ],
    compiler_params=pltpu.CompilerParams(needs_layout_passes=False),
)(_sc_body)


def kernel(x, emb_table, W, b):
    table_pad = jnp.pad(emb_table, ((0, VPAD - VOCAB), (0, 0)))
    wt = W.reshape(DIM, 1)
    b2 = b.reshape(1, 1)
    v = pl.pallas_call(
        _proj_kernel,
        out_shape=jax.ShapeDtypeStruct((VPAD, 1), jnp.float32),
    )(table_pad, wt, b2)
    out = _sc_call(v.reshape(VPAD), x.reshape(BATCH * SEQ))
    return out.reshape(BATCH, 1)


# speedup vs baseline: 185.8626x; 1.4637x over previous
import functools

import jax
import jax.numpy as jnp
from jax import lax
from jax.experimental import pallas as pl
from jax.experimental.pallas import tpu as pltpu
from jax.experimental.pallas import tpu_sc as plsc

VOCAB = 1000
VPAD = 1024
BATCH = 16384
SEQ = 200
DIM = 64

NC = 2
NS = 16
NW = NC * NS
LANES = 16
ROWS_PER_TILE = BATCH // NW
GROUPS_PER_TILE = ROWS_PER_TILE // LANES


def _proj_kernel(t_ref, wt_ref, b_ref, o_ref):
    o_ref[...] = (
        jnp.dot(t_ref[...], wt_ref[...], preferred_element_type=jnp.float32)
        + b_ref[...]
    )


def _sc_body(v_hbm, x_hbm, out_hbm, v_v, x_v, out_v):
    wid = lax.axis_index("s") * NC + lax.axis_index("c")
    row0 = wid * ROWS_PER_TILE
    base = row0 * SEQ

    pltpu.sync_copy(v_hbm, v_v)
    pltpu.sync_copy(x_hbm.at[pl.ds(base, ROWS_PER_TILE * SEQ)], x_v)

    lane_off = lax.iota(jnp.int32, 16) * SEQ

    UNROLL = 8

    def group_body(g, _):
        gbase = g * (LANES * SEQ)

        def tok_body(i, accs):
            accs = list(accs)
            t0 = gbase + i * UNROLL
            for j in range(UNROLL):
                offs = lane_off + (t0 + j)
                idxs = plsc.load_gather(x_v, [offs])
                vals = plsc.load_gather(v_v, [idxs])
                accs[j % 4] = accs[j % 4] + vals
            return tuple(accs)

        zero = jnp.zeros((16,), jnp.float32)
        a = lax.fori_loop(0, SEQ // UNROLL, tok_body, (zero, zero, zero, zero))
        acc = (a[0] + a[1]) + (a[2] + a[3])
        z = acc * (1.0 / SEQ)
        y = 1.0 / (1.0 + jnp.exp(-z))
        out_v[pl.ds(g * LANES, LANES)] = y
        return 0

    lax.fori_loop(0, GROUPS_PER_TILE, group_body, 0)

    pltpu.sync_copy(out_v, out_hbm.at[pl.ds(row0, ROWS_PER_TILE)])


_sc_call = functools.partial(
    pl.kernel,
    out_type=jax.ShapeDtypeStruct((BATCH,), jnp.float32),
    mesh=plsc.VectorSubcoreMesh(core_axis_name="c", subcore_axis_name="s"),
    scratch_types=[
        pltpu.VMEM((VPAD,), jnp.float32),
        pltpu.VMEM((ROWS_PER_TILE * SEQ,), jnp.int32),
        pltpu.VMEM((ROWS_PER_TILE,), jnp.float32),
    ],
    compiler_params=pltpu.CompilerParams(needs_layout_passes=False),
)(_sc_body)


def kernel(x, emb_table, W, b):
    table_pad = jnp.pad(emb_table, ((0, VPAD - VOCAB), (0, 0)))
    wt = W.reshape(DIM, 1)
    b2 = b.reshape(1, 1)
    v = pl.pallas_call(
        _proj_kernel,
        out_shape=jax.ShapeDtypeStruct((VPAD, 1), jnp.float32),
    )(table_pad, wt, b2)
    out = _sc_call(v.reshape(VPAD), x.reshape(BATCH * SEQ))
    return out.reshape(BATCH, 1)
